# baseline (speedup 1.0000x reference)
import functools

import jax
import jax.numpy as jnp
from jax.experimental import pallas as pl

_PATCH_SIZES = [4, 8, 12, 16, 24, 32, 48, 64]
_NUM_EXPERTS = 8


def _gelu(v):
    return 0.5 * v * (1.0 + jax.lax.erf(v * 0.7071067811865476))


def _moe_kernel(x_ref, gW1_ref, gb1_ref, gW2_ref, gb2_ref,
                iW1_ref, ib1_ref, iW2_ref, ib2_ref,
                rW1_ref, rb1_ref, rW2_ref, rb2_ref,
                wW1_ref, wb1_ref, wW2_ref, wb2_ref,
                out_ref, *, T):
    X = x_ref[0]
    D = X.shape[1]
    ctx = jnp.mean(X, axis=0, keepdims=True)

    minall = jnp.min(X, axis=0, keepdims=True)
    maxall = jnp.max(X, axis=0, keepdims=True)

    X34 = X.reshape(T // 4, 4, D)
    s1 = {4: jnp.sum(X34, axis=1)}
    s2 = {4: jnp.sum(X34 * X34, axis=1)}

    def _coarsen(src, g, npad):
        if npad:
            src = jnp.concatenate(
                [src, jnp.zeros((npad, src.shape[1]), src.dtype)], axis=0)
        n = src.shape[0] // g
        return jnp.sum(src.reshape(n, g, src.shape[1]), axis=1)

    for ps, base, g in [(8, 4, 2), (12, 4, 3), (16, 8, 2), (24, 12, 2),
                        (32, 16, 2), (48, 24, 2), (64, 32, 2)]:
        nb = s1[base].shape[0]
        npad = (-nb) % g
        s1[ps] = _coarsen(s1[base], g, npad)
        s2[ps] = _coarsen(s2[base], g, npad)

    feats = [ctx]
    pmeans = []
    offs = []
    off = 0
    for ps in _PATCH_SIZES:
        pad = (-T) % ps
        pm = s1[ps] * (1.0 / ps)
        var = (s2[ps] - s1[ps] * pm) * (1.0 / (ps - 1))
        std = jnp.sqrt(jnp.maximum(var, 0.0))
        stdmean = jnp.mean(std, axis=0, keepdims=True)
        if pad:
            fmin = jnp.minimum(minall, 0.0)
            fmax = jnp.maximum(maxall, 0.0)
        else:
            fmin, fmax = minall, maxall
        feats.extend([fmin, stdmean, fmax])
        pmeans.append(pm)
        offs.append(off)
        off += ps * D

    gate_in = jnp.concatenate(feats, axis=1)
    hg = _gelu(jnp.dot(gate_in, gW1_ref[...], preferred_element_type=jnp.float32)
               + gb1_ref[...])
    logits = (jnp.dot(hg, gW2_ref[...], preferred_element_type=jnp.float32)
              + gb2_ref[...])

    iota = jax.lax.broadcasted_iota(jnp.int32, logits.shape, 1)
    m1 = jnp.max(logits)
    i1 = jnp.min(jnp.where(logits == m1, iota, _NUM_EXPERTS))
    rest = jnp.where(iota == i1, -3e38, logits)
    m2 = jnp.max(rest)
    i2 = jnp.min(jnp.where(rest == m2, iota, _NUM_EXPERTS))
    g1 = 1.0 / (1.0 + jnp.exp(m2 - m1))
    g2 = 1.0 - g1
    coefs = (jnp.where(iota == i1, g1, 0.0)
             + jnp.where(iota == i2, g2, 0.0))

    out_ref[0] = X

    for e, ps in enumerate(_PATCH_SIZES):
        pad = (-T) % ps
        Tp = T + pad
        N = Tp // ps

        @pl.when(coefs[0, e] != 0.0)
        def _(e=e, ps=ps, pad=pad, Tp=Tp, N=N, off=offs[e], pm=pmeans[e],
              ce=coefs[0, e]):
            if pad:
                Xp = jnp.concatenate([X, jnp.zeros((pad, D), X.dtype)], axis=0)
            else:
                Xp = X
            X3 = Xp.reshape(N, ps, D)
            Xw = jnp.concatenate([X3[:, p, :] for p in range(ps)], axis=1)
            h = (jnp.dot(Xw, iW1_ref[off:off + ps * D, :],
                         preferred_element_type=jnp.float32)
                 + ib1_ref[e:e + 1])
            iw = (jnp.dot(h, iW2_ref[:, off:off + ps * D],
                          preferred_element_type=jnp.float32)
                  + ib2_ref[0:1, off:off + ps * D])
            hr = (jnp.dot(pm, rW1_ref[e], preferred_element_type=jnp.float32)
                  + rb1_ref[e:e + 1])
            inter = (jnp.dot(hr, rW2_ref[e], preferred_element_type=jnp.float32)
                     + rb2_ref[e:e + 1])
            hw = _gelu(jnp.dot(ctx, wW1_ref[e],
                               preferred_element_type=jnp.float32)
                       + wb1_ref[e:e + 1])
            wl = (jnp.sum(hw * wW2_ref[e:e + 1], axis=-1, keepdims=True)
                  + wb2_ref[e, 0])
            w = jax.nn.sigmoid(wl)[0, 0]
            pieces = [
                (w * iw[:, p * D:(p + 1) * D] + (1.0 - w) * inter)[:, None, :]
                for p in range(ps)
            ]
            fused = jnp.concatenate(pieces, axis=1).reshape(Tp, D)[:T]
            out_ref[0] = out_ref[0] + ce * fused


@jax.jit
def kernel(x, gate_params, expert_params):
    B, T, D = x.shape
    gW1, gb1, gW2, gb2 = gate_params

    iW1c = jnp.concatenate([p[0] for p in expert_params], axis=0)
    ib1s = jnp.stack([p[1] for p in expert_params])
    iW2c = jnp.concatenate([p[2] for p in expert_params], axis=1)
    ib2c = jnp.concatenate([p[3] for p in expert_params])[None, :]
    rW1s = jnp.stack([p[4] for p in expert_params])
    rb1s = jnp.stack([p[5] for p in expert_params])
    rW2s = jnp.stack([p[6] for p in expert_params])
    rb2s = jnp.stack([p[7] for p in expert_params])
    wW1s = jnp.stack([p[8] for p in expert_params])
    wb1s = jnp.stack([p[9] for p in expert_params])
    wW2s = jnp.stack([p[10][:, 0] for p in expert_params])
    wb2s = jnp.stack([p[11] for p in expert_params])

    weights = [gW1, gb1[None, :], gW2, gb2[None, :],
               iW1c, ib1s, iW2c, ib2c,
               rW1s, rb1s, rW2s, rb2s,
               wW1s, wb1s, wW2s, wb2s]

    def full(a):
        nd = a.ndim
        return pl.BlockSpec(a.shape, lambda b, _nd=nd: (0,) * _nd)

    return pl.pallas_call(
        functools.partial(_moe_kernel, T=T),
        grid=(B,),
        in_specs=[pl.BlockSpec((1, T, D), lambda b: (b, 0, 0))]
                 + [full(a) for a in weights],
        out_specs=pl.BlockSpec((1, T, D), lambda b: (b, 0, 0)),
        out_shape=jax.ShapeDtypeStruct((B, T, D), x.dtype),
    )(x, *weights)


# speedup vs baseline: 1.1643x; 1.1643x over previous
import functools

import jax
import jax.numpy as jnp
from jax.experimental import pallas as pl

_PATCH_SIZES = [4, 8, 12, 16, 24, 32, 48, 64]
_NUM_EXPERTS = 8
_T = 2048
_NS = [-(-_T // ps) for ps in _PATCH_SIZES]
_PM_OFFS = [sum(_NS[:i]) for i in range(len(_NS))]
_PM_ROWS = sum(_NS) + 2


def _gelu(v):
    return 0.5 * v * (1.0 + jax.lax.erf(v * 0.7071067811865476))


def _gate_kernel(x_ref, gW1_ref, gb1_ref, gW2_ref, gb2_ref, pm_ref, *, T, C):
    D = x_ref.shape[2]

    _CHUNK = 1024
    s1p, s2p, mnp, mxp = [], [], [], []
    for ci in range(C):
        for k in range(0, T, _CHUNK):
            c = x_ref[ci, k:k + _CHUNK, :]
            c3 = c.reshape(_CHUNK // 4, 4, D)
            s1p.append(jnp.sum(c3, axis=1))
            s2p.append(jnp.sum((c * c).reshape(_CHUNK // 4, 4, D), axis=1))
            mnp.append(jnp.min(c, axis=0, keepdims=True))
            mxp.append(jnp.max(c, axis=0, keepdims=True))
    nch = T // _CHUNK
    minv = jnp.min(jnp.concatenate(mnp, axis=0).reshape(C, nch, D), axis=1)
    maxv = jnp.max(jnp.concatenate(mxp, axis=0).reshape(C, nch, D), axis=1)
    s1 = {4: jnp.concatenate(s1p, axis=0)}
    s2 = {4: jnp.concatenate(s2p, axis=0)}

    def _coarsen(src, g, n_in, n_out):
        npad = n_out * g - n_in
        if npad:
            s3 = src.reshape(C, n_in, D)
            s3 = jnp.concatenate(
                [s3, jnp.zeros((C, npad, D), src.dtype)], axis=1)
            src = s3.reshape(C * n_out * g, D)
        return jnp.sum(src.reshape(C * n_out, g, D), axis=1)

    lvl = {4: T // 4}
    for ps, base, g in [(8, 4, 2), (12, 4, 3), (16, 8, 2), (24, 12, 2),
                        (32, 16, 2), (48, 24, 2), (64, 32, 2)]:
        n_in = lvl[base]
        n_out = -(-n_in // g)
        lvl[ps] = n_out
        s1[ps] = _coarsen(s1[base], g, n_in, n_out)
        s2[ps] = _coarsen(s2[base], g, n_in, n_out)

    ctx = jnp.sum(s1[64].reshape(C, lvl[64], D), axis=1) * (1.0 / T)

    feats = [ctx]
    for i, ps in enumerate(_PATCH_SIZES):
        n = lvl[ps]
        pm = s1[ps] * (1.0 / ps)
        var = (s2[ps] - s1[ps] * pm) * (1.0 / (ps - 1))
        std = jnp.sqrt(jnp.maximum(var, 0.0))
        stdmean = jnp.mean(std.reshape(C, n, D), axis=1)
        if T % ps:
            fmin = jnp.minimum(minv, 0.0)
            fmax = jnp.maximum(maxv, 0.0)
        else:
            fmin, fmax = minv, maxv
        feats.extend([fmin, stdmean, fmax])
        pm_ref[:, _PM_OFFS[i]:_PM_OFFS[i] + n, :] = pm.reshape(C, n, D)

    gate_in = jnp.concatenate(feats, axis=1)
    hg = _gelu(jnp.dot(gate_in, gW1_ref[...], preferred_element_type=jnp.float32)
               + gb1_ref[...])
    logits = (jnp.dot(hg, gW2_ref[...], preferred_element_type=jnp.float32)
              + gb2_ref[...])

    iota = jax.lax.broadcasted_iota(jnp.int32, logits.shape, 1)
    m1 = jnp.max(logits, axis=1, keepdims=True)
    i1 = jnp.min(jnp.where(logits == m1, iota, _NUM_EXPERTS), axis=1,
                 keepdims=True)
    rest = jnp.where(iota == i1, -3e38, logits)
    m2 = jnp.max(rest, axis=1, keepdims=True)
    i2 = jnp.min(jnp.where(rest == m2, iota, _NUM_EXPERTS), axis=1,
                 keepdims=True)
    g1 = 1.0 / (1.0 + jnp.exp(m2 - m1))
    coefs = (jnp.where(iota == i1, g1, 0.0)
             + jnp.where(iota == i2, 1.0 - g1, 0.0))
    cpad = jnp.concatenate(
        [coefs, jnp.zeros((C, D - _NUM_EXPERTS), coefs.dtype)], axis=1)

    pm_ref[:, _PM_ROWS - 2, :] = ctx
    pm_ref[:, _PM_ROWS - 1, :] = cpad


def _expert_kernel(x_ref, pm_ref,
                   iW1_ref, ib1_ref, iW2_ref, ib2_ref,
                   rW1_ref, rb1_ref, rW2_ref, rb2_ref,
                   wW1_ref, wb1_ref, wW2_ref, wb2_ref,
                   out_ref, *, T):
    X = x_ref[0]
    D = X.shape[1]
    PM = pm_ref[0]
    ctx = PM[_PM_ROWS - 2:_PM_ROWS - 1, :]
    crow = PM[_PM_ROWS - 1:_PM_ROWS, :]

    out_ref[0] = X

    off = 0
    for e, ps in enumerate(_PATCH_SIZES):
        pad = (-T) % ps
        Tp = T + pad
        N = Tp // ps

        @pl.when(crow[0, e] != 0.0)
        def _(e=e, ps=ps, pad=pad, Tp=Tp, N=N, off=off,
              pm=PM[_PM_OFFS[e]:_PM_OFFS[e] + N, :], ce=crow[0, e]):
            if pad:
                Xp = jnp.concatenate([X, jnp.zeros((pad, D), X.dtype)], axis=0)
            else:
                Xp = X
            X3 = Xp.reshape(N, ps, D)
            Xw = jnp.concatenate([X3[:, p, :] for p in range(ps)], axis=1)
            h = (jnp.dot(Xw, iW1_ref[off:off + ps * D, :],
                         preferred_element_type=jnp.float32)
                 + ib1_ref[e:e + 1])
            iw = (jnp.dot(h, iW2_ref[:, off:off + ps * D],
                          preferred_element_type=jnp.float32)
                  + ib2_ref[0:1, off:off + ps * D])
            hr = (jnp.dot(pm, rW1_ref[e], preferred_element_type=jnp.float32)
                  + rb1_ref[e:e + 1])
            inter = (jnp.dot(hr, rW2_ref[e], preferred_element_type=jnp.float32)
                     + rb2_ref[e:e + 1])
            hw = _gelu(jnp.dot(ctx, wW1_ref[e],
                               preferred_element_type=jnp.float32)
                       + wb1_ref[e:e + 1])
            wl = (jnp.sum(hw * wW2_ref[e:e + 1], axis=-1, keepdims=True)
                  + wb2_ref[e, 0])
            w = jax.nn.sigmoid(wl)[0, 0]
            pieces = [
                (w * iw[:, p * D:(p + 1) * D] + (1.0 - w) * inter)[:, None, :]
                for p in range(ps)
            ]
            fused = jnp.concatenate(pieces, axis=1).reshape(Tp, D)[:T]
            out_ref[0] = out_ref[0] + ce * fused
        off += ps * D


@jax.jit
def kernel(x, gate_params, expert_params):
    B, T, D = x.shape
    gW1, gb1, gW2, gb2 = gate_params
    C = 4 if B % 4 == 0 else 1

    pmall = pl.pallas_call(
        functools.partial(_gate_kernel, T=T, C=C),
        grid=(B // C,),
        in_specs=[pl.BlockSpec((C, T, D), lambda i: (i, 0, 0)),
                  pl.BlockSpec(gW1.shape, lambda i: (0, 0)),
                  pl.BlockSpec((1, D), lambda i: (0, 0)),
                  pl.BlockSpec(gW2.shape, lambda i: (0, 0)),
                  pl.BlockSpec((1, _NUM_EXPERTS), lambda i: (0, 0))],
        out_specs=pl.BlockSpec((C, _PM_ROWS, D), lambda i: (i, 0, 0)),
        out_shape=jax.ShapeDtypeStruct((B, _PM_ROWS, D), x.dtype),
    )(x, gW1, gb1[None, :], gW2, gb2[None, :])

    iW1c = jnp.concatenate([p[0] for p in expert_params], axis=0)
    ib1s = jnp.stack([p[1] for p in expert_params])
    iW2c = jnp.concatenate([p[2] for p in expert_params], axis=1)
    ib2c = jnp.concatenate([p[3] for p in expert_params])[None, :]
    rW1s = jnp.stack([p[4] for p in expert_params])
    rb1s = jnp.stack([p[5] for p in expert_params])
    rW2s = jnp.stack([p[6] for p in expert_params])
    rb2s = jnp.stack([p[7] for p in expert_params])
    wW1s = jnp.stack([p[8] for p in expert_params])
    wb1s = jnp.stack([p[9] for p in expert_params])
    wW2s = jnp.stack([p[10][:, 0] for p in expert_params])
    wb2s = jnp.stack([p[11] for p in expert_params])

    weights = [iW1c, ib1s, iW2c, ib2c,
               rW1s, rb1s, rW2s, rb2s,
               wW1s, wb1s, wW2s, wb2s]

    def full(a):
        nd = a.ndim
        return pl.BlockSpec(a.shape, lambda b, _nd=nd: (0,) * _nd)

    return pl.pallas_call(
        functools.partial(_expert_kernel, T=T),
        grid=(B,),
        in_specs=[pl.BlockSpec((1, T, D), lambda b: (b, 0, 0)),
                  pl.BlockSpec((1, _PM_ROWS, D), lambda b: (b, 0, 0))]
                 + [full(a) for a in weights],
        out_specs=pl.BlockSpec((1, T, D), lambda b: (b, 0, 0)),
        out_shape=jax.ShapeDtypeStruct((B, T, D), x.dtype),
    )(x, pmall, *weights)
